# baseline (speedup 1.0000x reference)
import functools

import jax
import jax.numpy as jnp
from jax.experimental import pallas as pl
from jax.experimental.pallas import tpu as pltpu

_E = 64
_K = 8
_COEF = 0.01


def _router_body(x_ref, w_ref, b_ref, rw_ref, se_ref, loss_ref,
                 lg_ref, acc_ref, *, n_tokens, hc):
    t = pl.program_id(0)
    h = pl.program_id(1)
    nt = pl.num_programs(0)
    nh = pl.num_programs(1)

    x = x_ref[...]
    w = w_ref[:, pl.ds(h * hc, hc)]
    part = jax.lax.dot_general(x, w, (((1,), (1,)), ((), ())),
                               preferred_element_type=jnp.float32)

    @pl.when(h == 0)
    def _first():
        lg_ref[...] = part + b_ref[...]

    @pl.when(h > 0)
    def _accum():
        lg_ref[...] += part

    @pl.when(h == nh - 1)
    def _route():
        logits = lg_ref[...]
        m = jnp.max(logits, axis=-1, keepdims=True)
        ex = jnp.exp(logits - m)
        scores = ex / jnp.sum(ex, axis=-1, keepdims=True)

        iota = jax.lax.broadcasted_iota(jnp.int32, scores.shape, 1)
        s = scores
        vals, idxs = [], []
        for _ in range(_K):
            mk = jnp.max(s, axis=-1, keepdims=True)
            ik = jnp.min(jnp.where(s == mk, iota, _E), axis=-1,
                         keepdims=True)
            vals.append(mk)
            idxs.append(ik)
            s = jnp.where(iota == ik, -1.0, s)
        topv = jnp.concatenate(vals, axis=-1)
        topi = jnp.concatenate(idxs, axis=-1)

        mm = jnp.max(topv, axis=-1, keepdims=True)
        e2 = jnp.exp(topv - mm)
        rw_ref[...] = e2 / jnp.sum(e2, axis=-1, keepdims=True)
        se_ref[...] = topi

        p_part = jnp.sum(scores, axis=0, keepdims=True)
        c_part = jnp.sum((s < 0.0).astype(jnp.float32), axis=0,
                         keepdims=True)

        @pl.when(t == 0)
        def _init():
            acc_ref[...] = jnp.zeros_like(acc_ref)

        acc_ref[0:1, :] += p_part
        acc_ref[1:2, :] += c_part

        @pl.when(t == nt - 1)
        def _fin():
            p_i = acc_ref[0:1, :] / n_tokens
            f_i = acc_ref[1:2, :] / (n_tokens * _K)
            loss_ref[0, 0] = _COEF * _E * jnp.sum(f_i * p_i)


def kernel(hidden_states, W, b):
    B, S, H = hidden_states.shape
    N = B * S
    x = hidden_states.reshape(N, H)
    T = min(2048, N)
    HC = min(2048, H)
    grid = (N // T, H // HC)
    rw, se, loss = pl.pallas_call(
        functools.partial(_router_body, n_tokens=float(N), hc=HC),
        grid=grid,
        in_specs=[
            pl.BlockSpec((T, HC), lambda t, h: (t, h)),
            pl.BlockSpec((_E, H), lambda t, h: (0, 0)),
            pl.BlockSpec((1, _E), lambda t, h: (0, 0)),
        ],
        out_specs=[
            pl.BlockSpec((T, _K), lambda t, h: (t, 0)),
            pl.BlockSpec((T, _K), lambda t, h: (t, 0)),
            pl.BlockSpec((1, 1), lambda t, h: (0, 0),
                         memory_space=pltpu.SMEM),
        ],
        out_shape=[
            jax.ShapeDtypeStruct((N, _K), jnp.float32),
            jax.ShapeDtypeStruct((N, _K), jnp.int32),
            jax.ShapeDtypeStruct((1, 1), jnp.float32),
        ],
        scratch_shapes=[
            pltpu.VMEM((T, _E), jnp.float32),
            pltpu.VMEM((2, _E), jnp.float32),
        ],
    )(x, W, b.reshape(1, _E))
    return rw.reshape(B, S, _K), se.reshape(B, S, _K), loss[0, 0]


# speedup vs baseline: 1.4016x; 1.4016x over previous
import functools

import jax
import jax.numpy as jnp
from jax.experimental import pallas as pl
from jax.experimental.pallas import tpu as pltpu

_E = 64
_K = 8
_COEF = 0.01


def _router_body(x_ref, w_ref, b_ref, rw_ref, se_ref, loss_ref, acc_ref,
                 *, n_tokens):
    i = pl.program_id(0)
    n = pl.num_programs(0)
    x = x_ref[...]
    w = w_ref[...]
    logits = jax.lax.dot_general(x, w, (((1,), (1,)), ((), ())),
                                 preferred_element_type=jnp.float32)
    logits = logits + b_ref[...]
    m = jnp.max(logits, axis=-1, keepdims=True)
    ex = jnp.exp(logits - m)
    scores = ex / jnp.sum(ex, axis=-1, keepdims=True)

    iota = jax.lax.broadcasted_iota(jnp.int32, scores.shape, 1)
    s = scores
    vals, idxs = [], []
    for _ in range(_K):
        mk = jnp.max(s, axis=-1, keepdims=True)
        ik = jnp.argmax(s, axis=-1, keepdims=True).astype(jnp.int32)
        vals.append(mk)
        idxs.append(ik)
        s = jnp.where(iota == ik, -1.0, s)
    topv = jnp.concatenate(vals, axis=-1)
    topi = jnp.concatenate(idxs, axis=-1)

    mm = jnp.max(topv, axis=-1, keepdims=True)
    e2 = jnp.exp(topv - mm)
    rw_ref[...] = e2 / jnp.sum(e2, axis=-1, keepdims=True)
    se_ref[...] = topi

    p_part = jnp.sum(scores, axis=0, keepdims=True)
    c_part = jnp.sum((s < 0.0).astype(jnp.float32), axis=0, keepdims=True)

    @pl.when(i == 0)
    def _init():
        acc_ref[...] = jnp.zeros_like(acc_ref)

    acc_ref[0:1, :] += p_part
    acc_ref[1:2, :] += c_part

    @pl.when(i == n - 1)
    def _fin():
        p_i = acc_ref[0:1, :] / n_tokens
        f_i = acc_ref[1:2, :] / (n_tokens * _K)
        loss_ref[0, 0] = _COEF * _E * jnp.sum(f_i * p_i)


def kernel(hidden_states, W, b):
    B, S, H = hidden_states.shape
    N = B * S
    x = hidden_states.reshape(N, H)
    T = min(1024, N)
    grid = (N // T,)
    rw, se, loss = pl.pallas_call(
        functools.partial(_router_body, n_tokens=float(N)),
        grid=grid,
        in_specs=[
            pl.BlockSpec((T, H), lambda i: (i, 0)),
            pl.BlockSpec((_E, H), lambda i: (0, 0)),
            pl.BlockSpec((1, _E), lambda i: (0, 0)),
        ],
        out_specs=[
            pl.BlockSpec((T, _K), lambda i: (i, 0)),
            pl.BlockSpec((T, _K), lambda i: (i, 0)),
            pl.BlockSpec((1, 1), lambda i: (0, 0), memory_space=pltpu.SMEM),
        ],
        out_shape=[
            jax.ShapeDtypeStruct((N, _K), jnp.float32),
            jax.ShapeDtypeStruct((N, _K), jnp.int32),
            jax.ShapeDtypeStruct((1, 1), jnp.float32),
        ],
        scratch_shapes=[pltpu.VMEM((2, _E), jnp.float32)],
    )(x, W, b.reshape(1, _E))
    return rw.reshape(B, S, _K), se.reshape(B, S, _K), loss[0, 0]
